# baseline (speedup 1.0000x reference)
import functools
import math

import jax
import jax.numpy as jnp
from jax.experimental import pallas as pl
from jax.experimental.pallas import tpu as pltpu


def _dilated_offsets(seq_len, include_local=2):
    offs = [0] + list(range(1, include_local + 1))
    k = 2
    while 2 ** k <= seq_len - 1:
        offs.append(2 ** k)
        k += 1
    return offs


def _projT_kernel(wq_ref, wk_ref, wv_ref, x_ref, o_ref):
    i = pl.program_id(0)
    dn = (((1,), (1,)), ((), ()))

    @pl.when(i == 0)
    def _():
        o_ref[...] = jax.lax.dot_general(
            wq_ref[...], x_ref[...], dn,
            preferred_element_type=jnp.float32).astype(jnp.bfloat16)

    @pl.when(i == 1)
    def _():
        o_ref[...] = jax.lax.dot_general(
            wk_ref[...], x_ref[...], dn,
            preferred_element_type=jnp.float32).astype(jnp.bfloat16)

    @pl.when(i == 2)
    def _():
        o_ref[...] = jax.lax.dot_general(
            wv_ref[...], x_ref[...], dn,
            preferred_element_type=jnp.float32).astype(jnp.bfloat16)


def _outproj_kernel(a_ref, w_ref, o_ref):
    dn = (((0,), (1,)), ((), ()))
    o_ref[...] = jax.lax.dot_general(a_ref[...], w_ref[...], dn,
                                     preferred_element_type=jnp.float32)


def _tree(terms, op):
    while len(terms) > 1:
        nxt = []
        for i in range(0, len(terms) - 1, 2):
            nxt.append(op(terms[i], terms[i + 1]))
        if len(terms) % 2:
            nxt.append(terms[-1])
        terms = nxt
    return terms[0]


def _attn_outproj_kernel(q_ref, k_ref, v_ref, wo_ref, o_ref,
                         at_ref, kpad_ref, vpad_ref,
                         *, offsets, scale, hd, H, BM):
    S = q_ref.shape[1]
    P = offsets[-1]
    t = pl.program_id(0)

    @pl.when(t == 0)
    def _():
        kpad_ref[:, 0:P] = jnp.zeros((hd, P), jnp.float32)
        vpad_ref[:, 0:P] = jnp.zeros((hd, P), jnp.float32)

    @pl.when(t < H)
    def _():
        kpad_ref[:, P:P + S] = k_ref[...].astype(jnp.float32)
        vpad_ref[:, P:P + S] = v_ref[...].astype(jnp.float32)
        q = q_ref[...].astype(jnp.float32)
        col = jax.lax.broadcasted_iota(jnp.int32, (1, S), 1)
        scores = []
        for o in offsets:
            s = jnp.sum(q * kpad_ref[:, P - o:P - o + S], axis=0,
                        keepdims=True) * scale
            scores.append(jnp.where(col >= o, s, -jnp.inf))
        m = _tree(scores, jnp.maximum)
        es = [jnp.exp(s - m) for s in scores]
        inv = 1.0 / _tree(es, jnp.add)
        terms = []
        for j, o in enumerate(offsets):
            terms.append((es[j] * inv) * vpad_ref[:, P - o:P - o + S])
        out = _tree(terms, jnp.add)
        at_ref[pl.ds(t * hd, hd), :] = out

    @pl.when(t >= H)
    def _():
        blk = t - H
        dn = (((0,), (1,)), ((), ()))
        a_blk = at_ref[:, pl.ds(blk * BM, BM)]
        o_ref[...] = jax.lax.dot_general(a_blk, wo_ref[...], dn,
                                         preferred_element_type=jnp.float32)


def kernel(x, Wq, Wk, Wv, Wo, positions, mask):
    B, S, D = x.shape
    H = 16
    hd = D // H
    scale = hd ** (-0.5)
    offsets = _dilated_offsets(S, 2)
    P = offsets[-1]

    BM = 512
    n_m = D // BM

    def _wsel(which):
        def im(i, m):
            return (jnp.where(i == which, m, 0), 0)
        return im

    projT = pl.pallas_call(
        _projT_kernel,
        grid=(3, n_m),
        in_specs=[
            pl.BlockSpec((BM, D), _wsel(0)),
            pl.BlockSpec((BM, D), _wsel(1)),
            pl.BlockSpec((BM, D), _wsel(2)),
            pl.BlockSpec((S, D), lambda i, m: (0, 0)),
        ],
        out_specs=pl.BlockSpec((BM, S), lambda i, m: (i * n_m + m, 0)),
        out_shape=jax.ShapeDtypeStruct((3 * D, S), jnp.bfloat16),
        compiler_params=pltpu.CompilerParams(
            dimension_semantics=("arbitrary", "arbitrary")),
    )

    nhb = D // hd
    n_s = S // BM

    def _hsel(off):
        def im(t):
            return (jnp.minimum(t, H - 1) + off, 0)
        return im

    attn_out = pl.pallas_call(
        functools.partial(_attn_outproj_kernel, offsets=offsets,
                          scale=scale, hd=hd, H=H, BM=BM),
        grid=(H + n_s,),
        in_specs=[
            pl.BlockSpec((hd, S), _hsel(0)),
            pl.BlockSpec((hd, S), _hsel(nhb)),
            pl.BlockSpec((hd, S), _hsel(2 * nhb)),
            pl.BlockSpec((D, D), lambda t: (0, 0)),
        ],
        out_specs=pl.BlockSpec(
            (BM, D), lambda t: (jnp.maximum(t - H, 0), 0)),
        out_shape=jax.ShapeDtypeStruct((S, D), jnp.float32),
        scratch_shapes=[
            pltpu.VMEM((D, S), jnp.float32),
            pltpu.VMEM((hd, P + S), jnp.float32),
            pltpu.VMEM((hd, P + S), jnp.float32),
        ],
        compiler_params=pltpu.CompilerParams(
            dimension_semantics=("arbitrary",)),
    )

    outs = []
    for b in range(B):
        xb = x[b]
        qkvT = projT(Wq, Wk, Wv, xb)
        outs.append(attn_out(qkvT, qkvT, qkvT, Wo))
    return jnp.stack(outs, axis=0)


# speedup vs baseline: 1.0238x; 1.0238x over previous
import functools
import math

import jax
import jax.numpy as jnp
from jax.experimental import pallas as pl
from jax.experimental.pallas import tpu as pltpu


def _dilated_offsets(seq_len, include_local=2):
    offs = [0] + list(range(1, include_local + 1))
    k = 2
    while 2 ** k <= seq_len - 1:
        offs.append(2 ** k)
        k += 1
    return offs


def _projT_kernel(wq_ref, wk_ref, wv_ref, x_ref, o_ref):
    i = pl.program_id(0)
    dn = (((1,), (1,)), ((), ()))

    @pl.when(i == 0)
    def _():
        o_ref[...] = jax.lax.dot_general(
            wq_ref[...], x_ref[...], dn,
            preferred_element_type=jnp.float32).astype(jnp.bfloat16)

    @pl.when(i == 1)
    def _():
        o_ref[...] = jax.lax.dot_general(
            wk_ref[...], x_ref[...], dn,
            preferred_element_type=jnp.float32).astype(jnp.bfloat16)

    @pl.when(i == 2)
    def _():
        o_ref[...] = jax.lax.dot_general(
            wv_ref[...], x_ref[...], dn,
            preferred_element_type=jnp.float32).astype(jnp.bfloat16)


def _outproj_kernel(a_ref, w_ref, o_ref):
    dn = (((0,), (1,)), ((), ()))
    o_ref[...] = jax.lax.dot_general(a_ref[...], w_ref[...], dn,
                                     preferred_element_type=jnp.float32)


def _tree(terms, op):
    while len(terms) > 1:
        nxt = []
        for i in range(0, len(terms) - 1, 2):
            nxt.append(op(terms[i], terms[i + 1]))
        if len(terms) % 2:
            nxt.append(terms[-1])
        terms = nxt
    return terms[0]


def _attn_outproj_kernel(q_ref, k_ref, v_ref, wo_ref, o_ref,
                         at_ref, kpad_ref, vpad_ref,
                         *, offsets, scale, hd, H, BM):
    S = q_ref.shape[1]
    P = offsets[-1]
    t = pl.program_id(0)

    @pl.when(t < H)
    def _():
        kpad_ref[:, 0:P] = jnp.zeros((hd, P), jnp.float32)
        vpad_ref[:, 0:P] = jnp.zeros((hd, P), jnp.float32)
        kpad_ref[:, P:P + S] = k_ref[...].astype(jnp.float32)
        vpad_ref[:, P:P + S] = v_ref[...].astype(jnp.float32)
        q = q_ref[...].astype(jnp.float32)
        col = jax.lax.broadcasted_iota(jnp.int32, (1, S), 1)
        scores = []
        for o in offsets:
            s = jnp.sum(q * kpad_ref[:, P - o:P - o + S], axis=0,
                        keepdims=True) * scale
            scores.append(jnp.where(col >= o, s, -jnp.inf))
        m = _tree(scores, jnp.maximum)
        es = [jnp.exp(s - m) for s in scores]
        inv = 1.0 / _tree(es, jnp.add)
        terms = []
        for j, o in enumerate(offsets):
            terms.append((es[j] * inv) * vpad_ref[:, P - o:P - o + S])
        out = _tree(terms, jnp.add)
        at_ref[pl.ds(t * hd, hd), :] = out

    @pl.when(t >= H)
    def _():
        blk = t - H
        dn = (((0,), (1,)), ((), ()))
        a_blk = at_ref[:, pl.ds(blk * BM, BM)]
        o_ref[...] = jax.lax.dot_general(a_blk, wo_ref[...], dn,
                                         preferred_element_type=jnp.float32)


def kernel(x, Wq, Wk, Wv, Wo, positions, mask):
    B, S, D = x.shape
    H = 16
    hd = D // H
    scale = hd ** (-0.5)
    offsets = _dilated_offsets(S, 2)
    P = offsets[-1]

    BM = 512
    n_m = D // BM

    def _wsel(which):
        def im(i, m):
            return (jnp.where(i == which, m, 0), 0)
        return im

    projT = pl.pallas_call(
        _projT_kernel,
        grid=(3, n_m),
        in_specs=[
            pl.BlockSpec((BM, D), _wsel(0)),
            pl.BlockSpec((BM, D), _wsel(1)),
            pl.BlockSpec((BM, D), _wsel(2)),
            pl.BlockSpec((S, D), lambda i, m: (0, 0)),
        ],
        out_specs=pl.BlockSpec((BM, S), lambda i, m: (i * n_m + m, 0)),
        out_shape=jax.ShapeDtypeStruct((3 * D, S), jnp.bfloat16),
        compiler_params=pltpu.CompilerParams(
            dimension_semantics=("arbitrary", "arbitrary")),
    )

    nhb = D // hd
    n_s = S // BM

    def _hsel(off):
        def im(t):
            return (jnp.minimum(t, H - 1) + off, 0)
        return im

    attn_out = pl.pallas_call(
        functools.partial(_attn_outproj_kernel, offsets=offsets,
                          scale=scale, hd=hd, H=H, BM=BM),
        grid=(H + n_s,),
        in_specs=[
            pl.BlockSpec((hd, S), _hsel(0)),
            pl.BlockSpec((hd, S), _hsel(nhb)),
            pl.BlockSpec((hd, S), _hsel(2 * nhb)),
            pl.BlockSpec((D, D), lambda t: (0, 0)),
        ],
        out_specs=pl.BlockSpec(
            (BM, D), lambda t: (jnp.maximum(t - H, 0), 0)),
        out_shape=jax.ShapeDtypeStruct((S, D), jnp.float32),
        scratch_shapes=[
            pltpu.VMEM((D, S), jnp.float32),
            pltpu.VMEM((hd, P + S), jnp.float32),
            pltpu.VMEM((hd, P + S), jnp.float32),
        ],
        compiler_params=pltpu.CompilerParams(
            dimension_semantics=("arbitrary",)),
    )

    outs = []
    for b in range(B):
        xb = x[b]
        qkvT = projT(Wq, Wk, Wv, xb)
        outs.append(attn_out(qkvT, qkvT, qkvT, Wo))
    return jnp.stack(outs, axis=0)
